# baseline (speedup 1.0000x reference)
import jax
import jax.numpy as jnp
from jax import lax
from jax.experimental import pallas as pl
from jax.experimental.pallas import tpu as pltpu
from jax.experimental.pallas import tpu_sc as plsc

NC = 2
NS = 16
NW = NC * NS
L = 16

CHUNK = 512
GROUPS = CHUNK // L
NCORN = 8
NIDX = NCORN * CHUNK


def _body(coords_hbm, t0_hbm, t1_hbm, t2_hbm, out_hbm,
          coords_v, idx_v, w_v, rows_v, out_v, sem):
    n = coords_hbm.shape[0] // 3
    per_w = n // NW
    n_chunks = per_w // CHUNK

    wid = lax.axis_index("s") * NC + lax.axis_index("c")
    tile_base = wid * per_w

    lanes = lax.iota(jnp.int32, L)
    offs = [dx * 16384 + dy * 128 + dz
            for dx in (0, 1) for dy in (0, 1) for dz in (0, 1)]

    @pl.loop(0, n_chunks)
    def _chunk(ci):
        base = tile_base + ci * CHUNK
        pltpu.sync_copy(coords_hbm.at[pl.ds(base * 3, CHUNK * 3)], coords_v)

        @pl.loop(0, GROUPS)
        def _groupA(g):
            p0 = pl.multiple_of(g * L, L)
            pts = (lanes + p0) * 3
            x = plsc.load_gather(coords_v, [pts])
            y = plsc.load_gather(coords_v, [pts + 1])
            z = plsc.load_gather(coords_v, [pts + 2])
            xs = x * 127.0
            ys = y * 127.0
            zs = z * 127.0
            i0 = jnp.minimum(xs.astype(jnp.int32), 126)
            j0 = jnp.minimum(ys.astype(jnp.int32), 126)
            k0 = jnp.minimum(zs.astype(jnp.int32), 126)
            fx = xs - i0.astype(jnp.float32)
            fy = ys - j0.astype(jnp.float32)
            fz = zs - k0.astype(jnp.float32)
            gx = 1.0 - fx
            gy = 1.0 - fy
            gz = 1.0 - fz
            ib = i0 * 16384 + j0 * 128 + k0
            wxy = (gx * gy, gx * fy, fx * gy, fx * fy)
            wz = (gz, fz)
            for k in range(NCORN):
                idx_v[pl.ds(k * CHUNK + p0, L)] = ib + offs[k]
                w_v[pl.ds(k * CHUNK + p0, L)] = wxy[k >> 1] * wz[k & 1]

        c0 = pltpu.async_copy(t0_hbm.at[idx_v], rows_v.at[pl.ds(0, NIDX)], sem)
        c1 = pltpu.async_copy(t1_hbm.at[idx_v],
                              rows_v.at[pl.ds(NIDX, NIDX)], sem)
        c2 = pltpu.async_copy(t2_hbm.at[idx_v],
                              rows_v.at[pl.ds(2 * NIDX, NIDX)], sem)
        c0.wait()
        c1.wait()
        c2.wait()

        @pl.loop(0, GROUPS)
        def _groupB(g):
            p0 = pl.multiple_of(g * L, L)
            opts = (lanes + p0) * 3
            w = [w_v[pl.ds(k * CHUNK + p0, L)] for k in range(NCORN)]
            for c in range(3):
                acc = w[0] * rows_v[pl.ds(c * NIDX + p0, L)]
                for k in range(1, NCORN):
                    v = rows_v[pl.ds(c * NIDX + k * CHUNK + p0, L)]
                    acc = acc + w[k] * v
                plsc.store_scatter(out_v, [opts + c], acc)

        pltpu.sync_copy(out_v, out_hbm.at[pl.ds(base * 3, CHUNK * 3)])


def kernel(coords, theta):
    n = coords.shape[0]
    nx, ny, nz, C = theta.shape
    v = nx * ny * nz
    planar = jnp.moveaxis(theta, -1, 0).reshape(C, v)

    mesh = plsc.VectorSubcoreMesh(core_axis_name="c", subcore_axis_name="s",
                                  num_cores=NC, num_subcores=NS)
    f = pl.kernel(
        _body,
        out_type=jax.ShapeDtypeStruct((n * C,), jnp.float32),
        mesh=mesh,
        compiler_params=pltpu.CompilerParams(needs_layout_passes=False),
        scratch_types=[
            pltpu.VMEM((CHUNK * 3,), jnp.float32),
            pltpu.VMEM((NIDX,), jnp.int32),
            pltpu.VMEM((NIDX,), jnp.float32),
            pltpu.VMEM((3 * NIDX,), jnp.float32),
            pltpu.VMEM((CHUNK * 3,), jnp.float32),
            pltpu.SemaphoreType.DMA,
        ],
    )
    out = f(coords.reshape(-1), planar[0], planar[1], planar[2])
    return out.reshape(n, C)
